# baseline (speedup 1.0000x reference)
import functools

import jax
import jax.numpy as jnp
from jax import lax
from jax.experimental import pallas as pl
from jax.experimental.pallas import tpu as pltpu
from jax.experimental.pallas import tpu_sc as plsc

D_IN = 1024
D_LAT = 32768
TOPK = 64
BATCH = 64
TL_ENC = 2048
TL_DEC = 2048

SC_NC = 2
SC_NS = 16
LANES = 16
N_WORKERS = SC_NC * SC_NS
ROWS_PER_W = BATCH // N_WORKERS
N_VEC = D_LAT // LANES


def _encode_body(x_ref, w_ref, b_ref, h_ref):
    acc = lax.dot_general(
        x_ref[...], w_ref[...], (((1,), (1,)), ((), ())),
        preferred_element_type=jnp.float32)
    h_ref[...] = jnp.maximum(acc + b_ref[...], 0.0)


def _zero_ref(ref, n):
    z = jnp.zeros((LANES,), jnp.int32)

    @plsc.parallel_loop(0, n // LANES, 1, unroll=4)
    def _(i):
        ref[pl.ds(i * LANES, LANES)] = z


def _vext(vec, pos):
    lane = lax.iota(jnp.int32, LANES)
    return jnp.sum(jnp.where(lane == pos, vec, 0))


def _select(coarse_ref, fine_ref, n_coarse_vec, target):
    lane = lax.iota(jnp.int32, LANES)

    def c_body(i, carry):
        run, fc, fbelow = carry
        v = coarse_ref[pl.ds(i * LANES, LANES)]
        pc = plsc.cumsum(v) + run
        pos = jnp.min(jnp.where(pc > target, lane, LANES))
        hit = pos < LANES
        pcat = _vext(pc, pos)
        vat = _vext(v, pos)
        first = jnp.logical_and(hit, fc < 0)
        fc = jnp.where(first, i * LANES + pos, fc)
        fbelow = jnp.where(first, pcat - vat, fbelow)
        run = jnp.max(pc)
        return run, fc, fbelow

    _, fc, fbelow = lax.fori_loop(
        0, n_coarse_vec, c_body, (jnp.int32(0), jnp.int32(-1), jnp.int32(0)))

    v = fine_ref[pl.ds(fc * LANES, LANES)]
    pc = plsc.cumsum(v) + fbelow
    pos = jnp.min(jnp.where(pc > target, lane, LANES))
    b = fc * LANES + pos
    at = _vext(pc, pos)
    below = at - _vext(v, pos)
    return b, below, at


UNROLL = 16


def _coarse_from_fine(fine_ref, coarse_ref, n_coarse_vec, lane):
    tot = jnp.zeros((LANES,), jnp.int32)
    for cv in range(n_coarse_vec):
        acc = jnp.zeros((LANES,), jnp.int32)
        for k in range(LANES):
            acc = acc + plsc.load_gather(
                fine_ref, [cv * 256 + lane * LANES + k])
        coarse_ref[pl.ds(cv * LANES, LANES)] = acc
        tot = tot + acc
    return jnp.sum(tot)


def _process_row(row_ref, h1_v, c1_v, h2_v, c2_v, h3_v, c3_v, lane, ones):
    _zero_ref(h1_v, 4096)
    _zero_ref(h2_v, 4096)
    _zero_ref(h3_v, 128)

    @plsc.parallel_loop(0, N_VEC, 1, unroll=UNROLL)
    def _(i):
        v = row_ref[pl.ds(i * LANES, LANES)]
        bits = lax.bitcast_convert_type(v, jnp.int32)
        plsc.addupdate_scatter(h1_v, [bits >> 19], ones, mask=bits > 0)

    n_pos = _coarse_from_fine(h1_v, c1_v, 16, lane)
    b1, _, at1 = _select(c1_v, h1_v, 16, n_pos - TOPK)
    above1 = n_pos - at1
    k1 = TOPK - above1

    b1s = lane * 0 + b1

    @plsc.parallel_loop(0, N_VEC, 1, unroll=UNROLL)
    def _(i):
        v = row_ref[pl.ds(i * LANES, LANES)]
        bits = lax.bitcast_convert_type(v, jnp.int32)
        m = jnp.logical_and((bits >> 19) == b1s, bits > 0)
        plsc.addupdate_scatter(h2_v, [(bits >> 7) & 0xFFF], ones, mask=m)

    n1 = _coarse_from_fine(h2_v, c2_v, 16, lane)
    b2, _, at2 = _select(c2_v, h2_v, 16, n1 - k1)
    above2 = n1 - at2
    k2 = k1 - above2

    t25 = lane * 0 + ((b1 << 12) | b2)

    @plsc.parallel_loop(0, N_VEC, 1, unroll=UNROLL)
    def _(i):
        v = row_ref[pl.ds(i * LANES, LANES)]
        bits = lax.bitcast_convert_type(v, jnp.int32)
        m = jnp.logical_and((bits >> 7) == t25, bits > 0)
        plsc.addupdate_scatter(h3_v, [bits & 0x7F], ones, mask=m)

    acc = jnp.zeros((LANES,), jnp.int32)
    for k in range(LANES):
        acc = acc + plsc.load_gather(h3_v, [(lane & 7) * LANES + k])
    acc = jnp.where(lane < 8, acc, 0)
    c3_v[pl.ds(0, LANES)] = acc
    n2 = jnp.sum(acc)
    b3, _, _ = _select(c3_v, h3_v, 1, n2 - k2)

    return (b1 << 19) | (b2 << 7) | b3


def _sc_thresh_body(h_hbm, thr_hbm, rowa_v, rowb_v, h1_v, c1_v, h2_v, c2_v,
                    h3_v, c3_v, out_v, sema, semb):
    wid = lax.axis_index("s") * SC_NC + lax.axis_index("c")
    lane = lax.iota(jnp.int32, LANES)
    ones = lane * 0 + 1

    row0 = wid * ROWS_PER_W
    cpa = pltpu.async_copy(h_hbm.at[row0], rowa_v, sema)
    cpb = pltpu.async_copy(h_hbm.at[row0 + 1], rowb_v, semb)
    cpa.wait()
    t0 = _process_row(rowa_v, h1_v, c1_v, h2_v, c2_v, h3_v, c3_v, lane, ones)
    cpb.wait()
    t1 = _process_row(rowb_v, h1_v, c1_v, h2_v, c2_v, h3_v, c3_v, lane, ones)

    tvec = jnp.where(lane == 0, lane * 0 + t0, lane * 0 + t1)
    out_v[...] = lax.bitcast_convert_type(tvec, jnp.float32)
    pltpu.sync_copy(out_v, thr_hbm.at[wid])


_sc_thresh = functools.partial(
    pl.kernel,
    out_type=jax.ShapeDtypeStruct((N_WORKERS, LANES), jnp.float32),
    mesh=plsc.VectorSubcoreMesh(core_axis_name="c", subcore_axis_name="s"),
    compiler_params=pltpu.CompilerParams(needs_layout_passes=False),
    scratch_types=[
        pltpu.VMEM((D_LAT,), jnp.float32),
        pltpu.VMEM((D_LAT,), jnp.float32),
        pltpu.VMEM((4096,), jnp.int32),
        pltpu.VMEM((256,), jnp.int32),
        pltpu.VMEM((4096,), jnp.int32),
        pltpu.VMEM((256,), jnp.int32),
        pltpu.VMEM((128,), jnp.int32),
        pltpu.VMEM((16,), jnp.int32),
        pltpu.VMEM((LANES,), jnp.float32),
        pltpu.SemaphoreType.DMA,
        pltpu.SemaphoreType.DMA,
    ],
)(_sc_thresh_body)


def _decode_body(h_ref, thr_ref, w_ref, b_ref, htopk_ref, xhat_ref):
    j = pl.program_id(0)
    h = h_ref[...]
    thr = thr_ref[:, :1]
    ht = jnp.where(h >= thr, h, 0.0)
    htopk_ref[...] = ht
    part = lax.dot_general(
        ht.astype(jnp.bfloat16), w_ref[...].astype(jnp.bfloat16),
        (((1,), (1,)), ((), ())),
        preferred_element_type=jnp.float32)

    @pl.when(j == 0)
    def _():
        xhat_ref[...] = part + b_ref[...]

    @pl.when(j != 0)
    def _():
        xhat_ref[...] += part


def kernel(x, W_enc, b_enc, W_dec, b_dec):
    b_enc2 = b_enc.reshape(1, D_LAT)
    b_dec2 = b_dec.reshape(1, D_IN)

    h = pl.pallas_call(
        _encode_body,
        grid=(D_LAT // TL_ENC,),
        in_specs=[
            pl.BlockSpec((BATCH, D_IN), lambda j: (0, 0)),
            pl.BlockSpec((TL_ENC, D_IN), lambda j: (j, 0)),
            pl.BlockSpec((1, TL_ENC), lambda j: (0, j)),
        ],
        out_specs=pl.BlockSpec((BATCH, TL_ENC), lambda j: (0, j)),
        out_shape=jax.ShapeDtypeStruct((BATCH, D_LAT), jnp.float32),
    )(x, W_enc, b_enc2)

    thr_sc = _sc_thresh(h)
    thr = jnp.broadcast_to(
        thr_sc[:, :ROWS_PER_W].reshape(BATCH, 1), (BATCH, 128))

    h_topk, x_hat = pl.pallas_call(
        _decode_body,
        grid=(D_LAT // TL_DEC,),
        in_specs=[
            pl.BlockSpec((BATCH, TL_DEC), lambda j: (0, j)),
            pl.BlockSpec((BATCH, 128), lambda j: (0, 0)),
            pl.BlockSpec((D_IN, TL_DEC), lambda j: (0, j)),
            pl.BlockSpec((1, D_IN), lambda j: (0, 0)),
        ],
        out_specs=[
            pl.BlockSpec((BATCH, TL_DEC), lambda j: (0, j)),
            pl.BlockSpec((BATCH, D_IN), lambda j: (0, 0)),
        ],
        out_shape=[
            jax.ShapeDtypeStruct((BATCH, D_LAT), jnp.float32),
            jax.ShapeDtypeStruct((BATCH, D_IN), jnp.float32),
        ],
        compiler_params=pltpu.CompilerParams(
            dimension_semantics=("arbitrary",)),
    )(h, thr, W_dec, b_dec2)

    return (x_hat, h, h_topk)


# speedup vs baseline: 1.3828x; 1.3828x over previous
import functools

import jax
import jax.numpy as jnp
from jax import lax
from jax.experimental import pallas as pl
from jax.experimental.pallas import tpu as pltpu
from jax.experimental.pallas import tpu_sc as plsc

D_IN = 1024
D_LAT = 32768
TOPK = 64
BATCH = 64
TL_ENC = 2048
TL_DEC = 2048

SC_NC = 2
SC_NS = 16
LANES = 16
N_WORKERS = SC_NC * SC_NS
ROWS_PER_W = BATCH // N_WORKERS
N_VEC = D_LAT // LANES


def _encode_body(x_ref, w_ref, b_ref, h_ref):
    acc = lax.dot_general(
        x_ref[...], w_ref[...], (((1,), (1,)), ((), ())),
        preferred_element_type=jnp.float32)
    h_ref[...] = jnp.maximum(acc + b_ref[...], 0.0)


def _zero_ref(ref, n):
    z = jnp.zeros((LANES,), jnp.int32)

    @plsc.parallel_loop(0, n // LANES, 1, unroll=4)
    def _(i):
        ref[pl.ds(i * LANES, LANES)] = z


def _vext(vec, pos):
    lane = lax.iota(jnp.int32, LANES)
    return jnp.sum(jnp.where(lane == pos, vec, 0))


def _select(coarse_ref, fine_ref, n_coarse_vec, target):
    lane = lax.iota(jnp.int32, LANES)

    def c_body(i, carry):
        run, fc, fbelow = carry
        v = coarse_ref[pl.ds(i * LANES, LANES)]
        pc = plsc.cumsum(v) + run
        pos = jnp.min(jnp.where(pc > target, lane, LANES))
        hit = pos < LANES
        pcat = _vext(pc, pos)
        vat = _vext(v, pos)
        first = jnp.logical_and(hit, fc < 0)
        fc = jnp.where(first, i * LANES + pos, fc)
        fbelow = jnp.where(first, pcat - vat, fbelow)
        run = jnp.max(pc)
        return run, fc, fbelow

    _, fc, fbelow = lax.fori_loop(
        0, n_coarse_vec, c_body, (jnp.int32(0), jnp.int32(-1), jnp.int32(0)))

    v = fine_ref[pl.ds(fc * LANES, LANES)]
    pc = plsc.cumsum(v) + fbelow
    pos = jnp.min(jnp.where(pc > target, lane, LANES))
    b = fc * LANES + pos
    at = _vext(pc, pos)
    below = at - _vext(v, pos)
    return b, below, at


UNROLL = 8


def _coarse_from_fine(fine_ref, coarse_ref, n_coarse_vec, lane):
    tot = jnp.zeros((LANES,), jnp.int32)
    for cv in range(n_coarse_vec):
        acc = jnp.zeros((LANES,), jnp.int32)
        for k in range(LANES):
            acc = acc + plsc.load_gather(
                fine_ref, [cv * 256 + lane * LANES + k])
        coarse_ref[pl.ds(cv * LANES, LANES)] = acc
        tot = tot + acc
    return jnp.sum(tot)


def _process_row(row_ref, h1_v, c1_v, h2_v, c2_v, h3_v, c3_v, lane, ones):
    _zero_ref(h1_v, 4096)
    _zero_ref(h2_v, 4096)
    _zero_ref(h3_v, 128)

    @plsc.parallel_loop(0, N_VEC, 1, unroll=UNROLL)
    def _(i):
        v = row_ref[pl.ds(i * LANES, LANES)]
        bits = lax.bitcast_convert_type(v, jnp.int32)
        plsc.addupdate_scatter(h1_v, [bits >> 19], ones, mask=bits > 0)

    n_pos = _coarse_from_fine(h1_v, c1_v, 16, lane)
    b1, _, at1 = _select(c1_v, h1_v, 16, n_pos - TOPK)
    above1 = n_pos - at1
    k1 = TOPK - above1

    b1s = lane * 0 + b1

    @plsc.parallel_loop(0, N_VEC, 1, unroll=UNROLL)
    def _(i):
        v = row_ref[pl.ds(i * LANES, LANES)]
        bits = lax.bitcast_convert_type(v, jnp.int32)
        m = jnp.logical_and((bits >> 19) == b1s, bits > 0)
        plsc.addupdate_scatter(h2_v, [(bits >> 7) & 0xFFF], ones, mask=m)

    n1 = _coarse_from_fine(h2_v, c2_v, 16, lane)
    b2, _, at2 = _select(c2_v, h2_v, 16, n1 - k1)
    above2 = n1 - at2
    k2 = k1 - above2

    t25 = lane * 0 + ((b1 << 12) | b2)

    @plsc.parallel_loop(0, N_VEC, 1, unroll=UNROLL)
    def _(i):
        v = row_ref[pl.ds(i * LANES, LANES)]
        bits = lax.bitcast_convert_type(v, jnp.int32)
        m = jnp.logical_and((bits >> 7) == t25, bits > 0)
        plsc.addupdate_scatter(h3_v, [bits & 0x7F], ones, mask=m)

    acc = jnp.zeros((LANES,), jnp.int32)
    for k in range(LANES):
        acc = acc + plsc.load_gather(h3_v, [(lane & 7) * LANES + k])
    acc = jnp.where(lane < 8, acc, 0)
    c3_v[pl.ds(0, LANES)] = acc
    n2 = jnp.sum(acc)
    b3, _, _ = _select(c3_v, h3_v, 1, n2 - k2)

    return (b1 << 19) | (b2 << 7) | b3


def _sc_thresh_body(h_hbm, thr_hbm, rowa_v, rowb_v, h1_v, c1_v, h2_v, c2_v,
                    h3_v, c3_v, out_v, sema, semb):
    wid = lax.axis_index("s") * SC_NC + lax.axis_index("c")
    lane = lax.iota(jnp.int32, LANES)
    ones = lane * 0 + 1

    row0 = wid * ROWS_PER_W
    cpa = pltpu.async_copy(h_hbm.at[row0], rowa_v, sema)
    cpb = pltpu.async_copy(h_hbm.at[row0 + 1], rowb_v, semb)
    cpa.wait()
    t0 = _process_row(rowa_v, h1_v, c1_v, h2_v, c2_v, h3_v, c3_v, lane, ones)
    cpb.wait()
    t1 = _process_row(rowb_v, h1_v, c1_v, h2_v, c2_v, h3_v, c3_v, lane, ones)

    tvec = jnp.where(lane == 0, lane * 0 + t0, lane * 0 + t1)
    out_v[...] = lax.bitcast_convert_type(tvec, jnp.float32)
    pltpu.sync_copy(out_v, thr_hbm.at[wid])


_sc_thresh = functools.partial(
    pl.kernel,
    out_type=jax.ShapeDtypeStruct((N_WORKERS, LANES), jnp.float32),
    mesh=plsc.VectorSubcoreMesh(core_axis_name="c", subcore_axis_name="s"),
    compiler_params=pltpu.CompilerParams(needs_layout_passes=False),
    scratch_types=[
        pltpu.VMEM((D_LAT,), jnp.float32),
        pltpu.VMEM((D_LAT,), jnp.float32),
        pltpu.VMEM((4096,), jnp.int32),
        pltpu.VMEM((256,), jnp.int32),
        pltpu.VMEM((4096,), jnp.int32),
        pltpu.VMEM((256,), jnp.int32),
        pltpu.VMEM((128,), jnp.int32),
        pltpu.VMEM((16,), jnp.int32),
        pltpu.VMEM((LANES,), jnp.float32),
        pltpu.SemaphoreType.DMA,
        pltpu.SemaphoreType.DMA,
    ],
)(_sc_thresh_body)


def _decode_body(h_ref, thr_ref, w_ref, b_ref, htopk_ref, xhat_ref):
    j = pl.program_id(0)
    h = h_ref[...]
    thr = thr_ref[:, :1]
    ht = jnp.where(h >= thr, h, 0.0)
    htopk_ref[...] = ht
    part = lax.dot_general(
        ht.astype(jnp.bfloat16), w_ref[...].astype(jnp.bfloat16),
        (((1,), (1,)), ((), ())),
        preferred_element_type=jnp.float32)

    @pl.when(j == 0)
    def _():
        xhat_ref[...] = part + b_ref[...]

    @pl.when(j != 0)
    def _():
        xhat_ref[...] += part


def kernel(x, W_enc, b_enc, W_dec, b_dec):
    b_enc2 = b_enc.reshape(1, D_LAT)
    b_dec2 = b_dec.reshape(1, D_IN)

    h = pl.pallas_call(
        _encode_body,
        grid=(D_LAT // TL_ENC,),
        in_specs=[
            pl.BlockSpec((BATCH, D_IN), lambda j: (0, 0)),
            pl.BlockSpec((TL_ENC, D_IN), lambda j: (j, 0)),
            pl.BlockSpec((1, TL_ENC), lambda j: (0, j)),
        ],
        out_specs=pl.BlockSpec((BATCH, TL_ENC), lambda j: (0, j)),
        out_shape=jax.ShapeDtypeStruct((BATCH, D_LAT), jnp.float32),
    )(x, W_enc, b_enc2)

    thr_sc = _sc_thresh(h)
    thr = jnp.broadcast_to(
        thr_sc[:, :ROWS_PER_W].reshape(BATCH, 1), (BATCH, 128))

    h_topk, x_hat = pl.pallas_call(
        _decode_body,
        grid=(D_LAT // TL_DEC,),
        in_specs=[
            pl.BlockSpec((BATCH, TL_DEC), lambda j: (0, j)),
            pl.BlockSpec((BATCH, 128), lambda j: (0, 0)),
            pl.BlockSpec((D_IN, TL_DEC), lambda j: (0, j)),
            pl.BlockSpec((1, D_IN), lambda j: (0, 0)),
        ],
        out_specs=[
            pl.BlockSpec((BATCH, TL_DEC), lambda j: (0, j)),
            pl.BlockSpec((BATCH, D_IN), lambda j: (0, 0)),
        ],
        out_shape=[
            jax.ShapeDtypeStruct((BATCH, D_LAT), jnp.float32),
            jax.ShapeDtypeStruct((BATCH, D_IN), jnp.float32),
        ],
        compiler_params=pltpu.CompilerParams(
            dimension_semantics=("arbitrary",)),
    )(h, thr, W_dec, b_dec2)

    return (x_hat, h, h_topk)
